# baseline (speedup 1.0000x reference)
import functools

import jax
import jax.numpy as jnp
from jax import lax
from jax.experimental import pallas as pl
from jax.experimental.pallas import tpu as pltpu
from jax.experimental.pallas import tpu_sc as plsc

N = 10000
D = 128
E = 320000
OUT = 3

NC = 2
NS = 16
NW = NC * NS
CHUNK = 128
NBUF = 3
CPW = 81
STEADY0 = 2
NGRP = 21
EPI0 = STEADY0 + NBUF * NGRP
E_PAD = NW * CPW * CHUNK
N_TRASH = 112
N_PAD = N + N_TRASH
RPT = N_PAD // NS

MBLK = 2000


def _make_sc_agg():
    mesh = plsc.VectorSubcoreMesh(core_axis_name="c", subcore_axis_name="s")

    @functools.partial(
        pl.kernel,
        out_type=jax.ShapeDtypeStruct((NC, N_PAD, D), jnp.float32),
        mesh=mesh,
        scratch_types=[
            pltpu.VMEM((NBUF, CHUNK), jnp.int32),
            pltpu.VMEM((NBUF, CHUNK), jnp.int32),
            pltpu.VMEM((NBUF, CHUNK, D), jnp.float32),
            pltpu.VMEM_SHARED((N_PAD, D), jnp.float32),
            pltpu.SemaphoreType.DMA((NBUF,)),
            pltpu.SemaphoreType.DMA((NBUF,)),
            pltpu.SemaphoreType.DMA((NBUF,)),
            pltpu.SemaphoreType.DMA((NBUF,)),
        ],
    )
    def sc_agg(src_hbm, dst_hbm, zeros_hbm, h_hbm, out_hbm,
               src_v, dst_v, rows_v, agg_sh, sem_i, sem_d, sem_g, sem_s):
        c = lax.axis_index("c")
        s = lax.axis_index("s")
        wid = c * NS + s

        def src_load(j, b):
            return pltpu.make_async_copy(
                src_hbm.at[pl.ds((wid * CPW + j) * CHUNK, CHUNK)],
                src_v.at[b], sem_i.at[b])

        def dst_load(j, b):
            return pltpu.make_async_copy(
                dst_hbm.at[pl.ds((wid * CPW + j) * CHUNK, CHUNK)],
                dst_v.at[b], sem_d.at[b])

        def gath(b):
            return pltpu.make_async_copy(
                h_hbm.at[src_v.at[b]], rows_v.at[b], sem_g.at[b])

        def scat(b):
            return pltpu.make_async_copy(
                rows_v.at[b], agg_sh.at[dst_v.at[b]], sem_s.at[b])

        def chunk(j, k, fill_src=True, issue_gather=True):
            b = k % NBUF
            bn = (k + 1) % NBUF
            gath(b).wait()
            if fill_src:
                src_load(j + 3, b).start()
            dst_load(j, b).wait()
            pltpu.async_copy(rows_v.at[b], agg_sh.at[dst_v.at[b]],
                             sem_s.at[b], add=True)
            if issue_gather:
                scat(bn).wait()
                src_load(j + 1, bn).wait()
                gath(bn).start()
                dst_load(j + 1, bn).start()

        for q in range(NBUF):
            src_load(q, q).start()
        dst_load(0, 0).start()
        pltpu.sync_copy(zeros_hbm, agg_sh.at[pl.ds(s * RPT, RPT)])
        plsc.subcore_barrier()
        src_load(0, 0).wait()
        gath(0).start()
        for j in (0, 1):
            b, bn = j % NBUF, (j + 1) % NBUF
            gath(b).wait()
            src_load(j + 3, b).start()
            dst_load(j, b).wait()
            pltpu.async_copy(rows_v.at[b], agg_sh.at[dst_v.at[b]],
                             sem_s.at[b], add=True)
            src_load(j + 1, bn).wait()
            gath(bn).start()
            dst_load(j + 1, bn).start()

        @pl.loop(0, NGRP)
        def _(g):
            base = g * NBUF + STEADY0
            for k in range(NBUF):
                chunk(base + k, STEADY0 + k)

        for j in range(EPI0, CPW):
            chunk(j, j, fill_src=(j + 3 < CPW), issue_gather=(j + 1 < CPW))
        for j in range(CPW - 3, CPW):
            scat(j % NBUF).wait()

        plsc.subcore_barrier()
        pltpu.sync_copy(agg_sh.at[pl.ds(s * RPT, RPT)],
                        out_hbm.at[c, pl.ds(s * RPT, RPT)])

    return sc_agg


_sc_agg = _make_sc_agg()


def _tc_body(relu, avg, h_ref, p_ref, ws_ref, wn_ref, b_ref, *rest):
    if avg:
        temp_ref, o_ref = rest
    else:
        (o_ref,) = rest
    agg = p_ref[0] + p_ref[1]
    y = (jnp.dot(h_ref[...], ws_ref[...], preferred_element_type=jnp.float32)
         + jnp.dot(agg, wn_ref[...], preferred_element_type=jnp.float32)
         + b_ref[...])
    if relu:
        y = jnp.maximum(y, 0.0)
    if avg:
        y = (temp_ref[...] + y) * 0.5
    o_ref[...] = y


def _make_tc(relu, avg):
    nblk = N // MBLK
    in_specs = [
        pl.BlockSpec((MBLK, D), lambda i: (i, 0)),
        pl.BlockSpec((NC, MBLK, D), lambda i: (0, i, 0)),
        pl.BlockSpec((D, D), lambda i: (0, 0)),
        pl.BlockSpec((D, D), lambda i: (0, 0)),
        pl.BlockSpec((1, D), lambda i: (0, 0)),
    ]
    if avg:
        in_specs.append(pl.BlockSpec((MBLK, D), lambda i: (i, 0)))
    return pl.pallas_call(
        functools.partial(_tc_body, relu, avg),
        grid=(nblk,),
        in_specs=in_specs,
        out_specs=pl.BlockSpec((MBLK, D), lambda i: (i, 0)),
        out_shape=jax.ShapeDtypeStruct((N, D), jnp.float32),
    )


_tc_relu = _make_tc(True, False)
_tc_relu_avg = _make_tc(True, True)
_tc_plain = _make_tc(False, False)


def kernel(edges, shape_features, Ws, Wn, bs, Wout_s, Wout_n, b_out):
    src = edges[0]
    dst = edges[1]
    pad = E_PAD - E
    pad_ids = lax.iota(jnp.int32, pad)
    src3 = jnp.concatenate([src, pad_ids % N])
    dst3 = jnp.concatenate([dst, N + (pad_ids % N_TRASH)])
    zeros = jnp.zeros((RPT, D), jnp.float32)

    def gconv(h, W_s, W_n, b, temp=None, relu=True):
        p = _sc_agg(src3, dst3, zeros, h)
        b2 = b.reshape(1, D)
        if temp is not None:
            return _tc_relu_avg(h, p, W_s, W_n, b2, temp)
        if relu:
            return _tc_relu(h, p, W_s, W_n, b2)
        return _tc_plain(h, p, W_s, W_n, b2)

    h = gconv(shape_features, Ws[0], Wn[0], bs[0])
    for i in range(1, 12, 2):
        temp = h
        h = gconv(h, Ws[i], Wn[i], bs[i])
        h = gconv(h, Ws[i + 1], Wn[i + 1], bs[i + 1], temp=temp)

    Wo_s = jnp.zeros((D, D), jnp.float32).at[:, :OUT].set(Wout_s)
    Wo_n = jnp.zeros((D, D), jnp.float32).at[:, :OUT].set(Wout_n)
    bo = jnp.zeros((D,), jnp.float32).at[:OUT].set(b_out)
    coords = gconv(h, Wo_s, Wo_n, bo, relu=False)[:, :OUT]
    return (h, coords)


# speedup vs baseline: 1.2245x; 1.2245x over previous
import functools

import jax
import jax.numpy as jnp
from jax import lax
from jax.experimental import pallas as pl
from jax.experimental.pallas import tpu as pltpu
from jax.experimental.pallas import tpu_sc as plsc

N = 10000
D = 128
E = 320000
OUT = 3

NC = 2
NS = 16
NW = NC * NS
CHUNK = 128
NBUF = 3
CPW = 81
STEADY0 = 1
NGRP = 25
EPI0 = STEADY0 + NBUF * NGRP
E_PAD = NW * CPW * CHUNK
N_TRASH = 112
N_PAD = N + N_TRASH
RPT = N_PAD // NS

MBLK = 2000


def _make_sc_agg():
    mesh = plsc.VectorSubcoreMesh(core_axis_name="c", subcore_axis_name="s")

    @functools.partial(
        pl.kernel,
        out_type=jax.ShapeDtypeStruct((NC, N_PAD, D), jnp.float32),
        mesh=mesh,
        scratch_types=[
            pltpu.VMEM((NBUF, CHUNK), jnp.int32),
            pltpu.VMEM((NBUF, CHUNK), jnp.int32),
            pltpu.VMEM((NBUF, CHUNK, D), jnp.float32),
            pltpu.VMEM_SHARED((N_PAD, D), jnp.float32),
            pltpu.SemaphoreType.DMA((NBUF,)),
            pltpu.SemaphoreType.DMA((NBUF,)),
            pltpu.SemaphoreType.DMA((NBUF,)),
            pltpu.SemaphoreType.DMA((NBUF,)),
        ],
    )
    def sc_agg(src_hbm, dst_hbm, zeros_hbm, h_hbm, out_hbm,
               src_v, dst_v, rows_v, agg_sh, sem_i, sem_d, sem_g, sem_s):
        c = lax.axis_index("c")
        s = lax.axis_index("s")
        wid = c * NS + s

        def src_load(j, b):
            return pltpu.make_async_copy(
                src_hbm.at[pl.ds((wid * CPW + j) * CHUNK, CHUNK)],
                src_v.at[b], sem_i.at[b])

        def dst_load(j, b):
            return pltpu.make_async_copy(
                dst_hbm.at[pl.ds((wid * CPW + j) * CHUNK, CHUNK)],
                dst_v.at[b], sem_d.at[b])

        def gath(b):
            return pltpu.make_async_copy(
                h_hbm.at[src_v.at[b]], rows_v.at[b], sem_g.at[b])

        def scat(b):
            return pltpu.make_async_copy(
                rows_v.at[b], agg_sh.at[dst_v.at[b]], sem_s.at[b])

        def chunk(j, k, fill_src=True, issue_gather=True):
            b = k % NBUF
            bp = (k + 2) % NBUF
            gath(b).wait()
            if fill_src:
                src_load(j + 3, b).start()
            dst_load(j, b).wait()
            pltpu.async_copy(rows_v.at[b], agg_sh.at[dst_v.at[b]],
                             sem_s.at[b], add=True)
            if issue_gather:
                scat(bp).wait()
                src_load(j + 2, bp).wait()
                gath(bp).start()
                dst_load(j + 2, bp).start()

        for q in range(NBUF):
            src_load(q, q).start()
            dst_load(q, q).start()
        pltpu.sync_copy(zeros_hbm, agg_sh.at[pl.ds(s * RPT, RPT)])
        plsc.subcore_barrier()
        for q in range(NBUF):
            src_load(q, q).wait()
            gath(q).start()
        gath(0).wait()
        src_load(3, 0).start()
        dst_load(0, 0).wait()
        pltpu.async_copy(rows_v.at[0], agg_sh.at[dst_v.at[0]],
                         sem_s.at[0], add=True)

        @pl.loop(0, NGRP)
        def _(g):
            base = g * NBUF + STEADY0
            for k in range(NBUF):
                chunk(base + k, STEADY0 + k)

        for j in range(EPI0, CPW):
            chunk(j, j, fill_src=(j + 3 < CPW), issue_gather=(j + 2 < CPW))
        for j in range(CPW - 3, CPW):
            scat(j % NBUF).wait()

        plsc.subcore_barrier()
        pltpu.sync_copy(agg_sh.at[pl.ds(s * RPT, RPT)],
                        out_hbm.at[c, pl.ds(s * RPT, RPT)])

    return sc_agg


_sc_agg = _make_sc_agg()


def _tc_body(relu, avg, h_ref, p_ref, ws_ref, wn_ref, b_ref, *rest):
    if avg:
        temp_ref, o_ref = rest
    else:
        (o_ref,) = rest
    agg = p_ref[0] + p_ref[1]
    y = (jnp.dot(h_ref[...], ws_ref[...], preferred_element_type=jnp.float32)
         + jnp.dot(agg, wn_ref[...], preferred_element_type=jnp.float32)
         + b_ref[...])
    if relu:
        y = jnp.maximum(y, 0.0)
    if avg:
        y = (temp_ref[...] + y) * 0.5
    o_ref[...] = y


def _make_tc(relu, avg):
    nblk = N // MBLK
    in_specs = [
        pl.BlockSpec((MBLK, D), lambda i: (i, 0)),
        pl.BlockSpec((NC, MBLK, D), lambda i: (0, i, 0)),
        pl.BlockSpec((D, D), lambda i: (0, 0)),
        pl.BlockSpec((D, D), lambda i: (0, 0)),
        pl.BlockSpec((1, D), lambda i: (0, 0)),
    ]
    if avg:
        in_specs.append(pl.BlockSpec((MBLK, D), lambda i: (i, 0)))
    return pl.pallas_call(
        functools.partial(_tc_body, relu, avg),
        grid=(nblk,),
        in_specs=in_specs,
        out_specs=pl.BlockSpec((MBLK, D), lambda i: (i, 0)),
        out_shape=jax.ShapeDtypeStruct((N, D), jnp.float32),
    )


_tc_relu = _make_tc(True, False)
_tc_relu_avg = _make_tc(True, True)
_tc_plain = _make_tc(False, False)


def kernel(edges, shape_features, Ws, Wn, bs, Wout_s, Wout_n, b_out):
    src = edges[0]
    dst = edges[1]
    pad = E_PAD - E
    pad_ids = lax.iota(jnp.int32, pad)
    src3 = jnp.concatenate([src, pad_ids % N])
    dst3 = jnp.concatenate([dst, N + (pad_ids % N_TRASH)])
    zeros = jnp.zeros((RPT, D), jnp.float32)

    def gconv(h, W_s, W_n, b, temp=None, relu=True):
        p = _sc_agg(src3, dst3, zeros, h)
        b2 = b.reshape(1, D)
        if temp is not None:
            return _tc_relu_avg(h, p, W_s, W_n, b2, temp)
        if relu:
            return _tc_relu(h, p, W_s, W_n, b2)
        return _tc_plain(h, p, W_s, W_n, b2)

    h = gconv(shape_features, Ws[0], Wn[0], bs[0])
    for i in range(1, 12, 2):
        temp = h
        h = gconv(h, Ws[i], Wn[i], bs[i])
        h = gconv(h, Ws[i + 1], Wn[i + 1], bs[i + 1], temp=temp)

    Wo_s = jnp.zeros((D, D), jnp.float32).at[:, :OUT].set(Wout_s)
    Wo_n = jnp.zeros((D, D), jnp.float32).at[:, :OUT].set(Wout_n)
    bo = jnp.zeros((D,), jnp.float32).at[:OUT].set(b_out)
    coords = gconv(h, Wo_s, Wo_n, bo, relu=False)[:, :OUT]
    return (h, coords)
